# baseline (speedup 1.0000x reference)
import functools

import jax
import jax.numpy as jnp
from jax import lax
from jax.experimental import pallas as pl
from jax.experimental.pallas import tpu as pltpu
from jax.experimental.pallas import tpu_sc as plsc

B = 128
V = 100000
TOP_K = 8
NW = 32
ROWS_PER_W = B // NW
CHUNK = 20000
NVEC = CHUNK // 16


def _sc_kernel(pred_hbm, ytrue_hbm, out_hbm, yt_v, win_v, buf_v, hit_v):
    nc = 2
    wid = lax.axis_index("s") * nc + lax.axis_index("c")
    lanes = lax.iota(jnp.int32, 16)

    pltpu.sync_copy(ytrue_hbm, yt_v)

    hits = jnp.float32(0.0)
    for i in range(ROWS_PER_W):
        r = wid * ROWS_PER_W + i
        yt_win = yt_v[pl.ds((r // 16) * 16, 16)]
        yt_scalar = jnp.max(jnp.where(lanes == r % 16, yt_win, 0))
        yt_splat = jnp.full((16,), yt_scalar, jnp.int32)

        start = jnp.minimum((yt_scalar // 8) * 8, V - 16)
        pltpu.sync_copy(pred_hbm.at[pl.ds(r * V + start, 16)], win_v)
        wv = win_v[...]
        t_scalar = jnp.max(
            jnp.where(lanes == yt_scalar - start, wv, jnp.float32(-3e38)))
        t_splat = jnp.full((16,), t_scalar, jnp.float32)

        rank = jnp.int32(0)
        for c in range(V // CHUNK):
            pltpu.sync_copy(pred_hbm.at[pl.ds(r * V + c * CHUNK, CHUNK)], buf_v)

            def body(j, carry):
                acc, idxv = carry
                v = buf_v[pl.ds(j * 16, 16)]
                m_gt = v > t_splat
                m_ge = v >= t_splat
                m_lt = idxv < yt_splat
                m = jnp.where(m_lt, m_ge, m_gt)
                return acc + m.astype(jnp.int32), idxv + 16

            acc0 = jnp.zeros((16,), jnp.int32)
            idx0 = lanes + c * CHUNK
            acc, _ = lax.fori_loop(0, NVEC, body, (acc0, idx0))
            rank = rank + jnp.sum(acc)
        hits = hits + (rank < TOP_K).astype(jnp.float32)

    hit_v[...] = jnp.full((16,), hits, jnp.float32)
    pltpu.sync_copy(hit_v, out_hbm.at[wid])


@jax.jit
def _topk_hits(pred_flat, ytrue):
    mesh = plsc.VectorSubcoreMesh(core_axis_name="c", subcore_axis_name="s")
    kern = functools.partial(
        pl.kernel,
        mesh=mesh,
        compiler_params=pltpu.CompilerParams(needs_layout_passes=False),
        out_type=jax.ShapeDtypeStruct((NW, 16), jnp.float32),
        scratch_types=[
            pltpu.VMEM((B,), jnp.int32),
            pltpu.VMEM((16,), jnp.float32),
            pltpu.VMEM((CHUNK,), jnp.float32),
            pltpu.VMEM((16,), jnp.float32),
        ],
    )(_sc_kernel)
    return kern(pred_flat, ytrue)


def kernel(y_pred, y_true):
    partial = _topk_hits(y_pred.reshape(-1), y_true.astype(jnp.int32))
    return partial[:, 0].sum() / jnp.float32(B)


# speedup vs baseline: 1.5603x; 1.5603x over previous
import functools

import jax
import jax.numpy as jnp
from jax import lax
from jax.experimental import pallas as pl
from jax.experimental.pallas import tpu as pltpu
from jax.experimental.pallas import tpu_sc as plsc

B = 128
V = 100000
TOP_K = 8
NW = 32
ROWS_PER_W = B // NW
CHUNK = 20000
NVEC = CHUNK // 16
CHUNKS_PER_ROW = V // CHUNK
NCHUNKS = ROWS_PER_W * CHUNKS_PER_ROW


def _sc_kernel(pred_hbm, ytrue_hbm, out_hbm, yt_v, bwin_v, buf0_v, buf1_v,
               hit_v, sem0, sem1, semb):
    nc = 2
    wid = lax.axis_index("s") * nc + lax.axis_index("c")
    lanes = lax.iota(jnp.int32, 16)
    r0 = wid * ROWS_PER_W
    bufs = (buf0_v, buf1_v)
    sems = (sem0, sem1)

    def chunk_src(k):
        r, c = k // CHUNKS_PER_ROW, k % CHUNKS_PER_ROW
        return pred_hbm.at[pl.ds((r0 + r) * V + c * CHUNK, CHUNK)]

    copies = {0: pltpu.async_copy(chunk_src(0), bufs[0], sems[0])}
    pltpu.sync_copy(ytrue_hbm, yt_v)

    ytw = yt_v[pl.ds((r0 // 16) * 16, 16)]
    yts, svs, tsplats, bcnts = [], [], [], []
    win_copies = []
    for i in range(ROWS_PER_W):
        lane_i = r0 - (r0 // 16) * 16 + i
        yt = jnp.max(jnp.where(lanes == lane_i, ytw, 0))
        sv = yt // 16
        win_copies.append(pltpu.async_copy(
            pred_hbm.at[pl.ds((r0 + i) * V + sv * 16, 16)],
            bwin_v.at[i], semb))
        yts.append(yt)
        svs.append(sv)
    for i in range(ROWS_PER_W):
        win_copies[i].wait()
        bv = bwin_v[i]
        q = jnp.full((16,), yts[i] - svs[i] * 16, jnp.int32)
        t_splat = jnp.full(
            (16,), jnp.max(jnp.where(lanes == q, bv, jnp.float32(-3e38))),
            jnp.float32)
        m_b = jnp.where(lanes < q, bv >= t_splat, bv > t_splat)
        tsplats.append(t_splat)
        bcnts.append(plsc.all_reduce_population_count(m_b))

    hits = jnp.float32(0.0)
    acc = jnp.zeros((16,), jnp.int32)
    for k in range(NCHUNKS):
        r, c = k // CHUNKS_PER_ROW, k % CHUNKS_PER_ROW
        if k + 1 < NCHUNKS:
            copies[k + 1] = pltpu.async_copy(
                chunk_src(k + 1), bufs[(k + 1) % 2], sems[(k + 1) % 2])
        copies[k].wait()
        buf = bufs[k % 2]
        if c == 0:
            acc = bcnts[r]
        t_splat = tsplats[r]
        lo = c * NVEC
        split1 = jnp.clip(svs[r] - lo, 0, NVEC)
        js = jnp.clip(svs[r] + 1 - lo, 0, NVEC)

        def _pre(j, a):
            return a + plsc.all_reduce_population_count(
                buf[pl.ds(j * 16, 16)] >= t_splat)

        def _suf(j, a):
            return a + plsc.all_reduce_population_count(
                buf[pl.ds(j * 16, 16)] > t_splat)

        acc = plsc.parallel_loop(0, split1, unroll=8, carry=acc)(_pre)
        acc = plsc.parallel_loop(js, NVEC, unroll=8, carry=acc)(_suf)
        if c == CHUNKS_PER_ROW - 1:
            rank = jnp.max(acc)
            hits = hits + (rank < TOP_K).astype(jnp.float32)

    hit_v[...] = jnp.full((16,), hits, jnp.float32)
    pltpu.sync_copy(hit_v, out_hbm.at[wid])


@jax.jit
def _topk_hits(pred_flat, ytrue):
    mesh = plsc.VectorSubcoreMesh(core_axis_name="c", subcore_axis_name="s")
    kern = functools.partial(
        pl.kernel,
        mesh=mesh,
        compiler_params=pltpu.CompilerParams(needs_layout_passes=False),
        out_type=jax.ShapeDtypeStruct((NW, 16), jnp.float32),
        scratch_types=[
            pltpu.VMEM((B,), jnp.int32),
            pltpu.VMEM((ROWS_PER_W, 16), jnp.float32),
            pltpu.VMEM((CHUNK,), jnp.float32),
            pltpu.VMEM((CHUNK,), jnp.float32),
            pltpu.VMEM((16,), jnp.float32),
            pltpu.SemaphoreType.DMA,
            pltpu.SemaphoreType.DMA,
            pltpu.SemaphoreType.DMA,
        ],
    )(_sc_kernel)
    return kern(pred_flat, ytrue)


def kernel(y_pred, y_true):
    partial = _topk_hits(y_pred.reshape(-1), y_true.astype(jnp.int32))
    return partial[:, 0].sum() / jnp.float32(B)


# speedup vs baseline: 1.6253x; 1.0417x over previous
import functools

import jax
import jax.numpy as jnp
from jax import lax
from jax.experimental import pallas as pl
from jax.experimental.pallas import tpu as pltpu
from jax.experimental.pallas import tpu_sc as plsc

B = 128
V = 100000
TOP_K = 8
NW = 32
ROWS_PER_W = B // NW
CHUNK = 20000
NVEC = CHUNK // 16
CHUNKS_PER_ROW = V // CHUNK
NCHUNKS = ROWS_PER_W * CHUNKS_PER_ROW


NBUF = 4


def _sc_kernel(pred_hbm, ytrue_hbm, out_hbm, yt_v, bwin_v, buf0_v, buf1_v,
               buf2_v, buf3_v, hit_v, sem0, sem1, sem2, sem3, semb):
    nc = 2
    wid = lax.axis_index("s") * nc + lax.axis_index("c")
    lanes = lax.iota(jnp.int32, 16)
    r0 = wid * ROWS_PER_W
    bufs = (buf0_v, buf1_v, buf2_v, buf3_v)
    sems = (sem0, sem1, sem2, sem3)

    def chunk_src(k):
        r, c = k // CHUNKS_PER_ROW, k % CHUNKS_PER_ROW
        return pred_hbm.at[pl.ds((r0 + r) * V + c * CHUNK, CHUNK)]

    copies = {k: pltpu.async_copy(chunk_src(k), bufs[k], sems[k])
              for k in range(NBUF - 1)}
    pltpu.sync_copy(ytrue_hbm, yt_v)

    ytw = yt_v[pl.ds((r0 // 16) * 16, 16)]
    yts, svs, tsplats, bcnts = [], [], [], []
    win_copies = []
    for i in range(ROWS_PER_W):
        lane_i = r0 - (r0 // 16) * 16 + i
        yt = jnp.max(jnp.where(lanes == lane_i, ytw, 0))
        sv = yt // 16
        win_copies.append(pltpu.async_copy(
            pred_hbm.at[pl.ds((r0 + i) * V + sv * 16, 16)],
            bwin_v.at[i], semb))
        yts.append(yt)
        svs.append(sv)
    for i in range(ROWS_PER_W):
        win_copies[i].wait()
        bv = bwin_v[i]
        q = jnp.full((16,), yts[i] - svs[i] * 16, jnp.int32)
        t_splat = jnp.full(
            (16,), jnp.max(jnp.where(lanes == q, bv, jnp.float32(-3e38))),
            jnp.float32)
        m_b = jnp.where(lanes < q, bv >= t_splat, bv > t_splat)
        tsplats.append(t_splat)
        bcnts.append(plsc.all_reduce_population_count(m_b))

    hits = jnp.float32(0.0)
    acc = jnp.zeros((16,), jnp.int32)
    for k in range(NCHUNKS):
        r, c = k // CHUNKS_PER_ROW, k % CHUNKS_PER_ROW
        kn = k + NBUF - 1
        if kn < NCHUNKS:
            copies[kn] = pltpu.async_copy(
                chunk_src(kn), bufs[kn % NBUF], sems[kn % NBUF])
        copies[k].wait()
        buf = bufs[k % NBUF]
        if c == 0:
            acc = bcnts[r]
        t_splat = tsplats[r]
        lo = c * NVEC
        split1 = jnp.clip(svs[r] - lo, 0, NVEC)
        js = jnp.clip(svs[r] + 1 - lo, 0, NVEC)

        def _pre(j, a):
            return a + plsc.all_reduce_population_count(
                buf[pl.ds(j * 16, 16)] >= t_splat)

        def _suf(j, a):
            return a + plsc.all_reduce_population_count(
                buf[pl.ds(j * 16, 16)] > t_splat)

        acc = plsc.parallel_loop(0, split1, unroll=8, carry=acc)(_pre)
        acc = plsc.parallel_loop(js, NVEC, unroll=8, carry=acc)(_suf)
        if c == CHUNKS_PER_ROW - 1:
            rank = jnp.max(acc)
            hits = hits + (rank < TOP_K).astype(jnp.float32)

    hit_v[...] = jnp.full((16,), hits, jnp.float32)
    pltpu.sync_copy(hit_v, out_hbm.at[wid])


@jax.jit
def _topk_hits(pred_flat, ytrue):
    mesh = plsc.VectorSubcoreMesh(core_axis_name="c", subcore_axis_name="s")
    kern = functools.partial(
        pl.kernel,
        mesh=mesh,
        compiler_params=pltpu.CompilerParams(needs_layout_passes=False),
        out_type=jax.ShapeDtypeStruct((NW, 16), jnp.float32),
        scratch_types=[
            pltpu.VMEM((B,), jnp.int32),
            pltpu.VMEM((ROWS_PER_W, 16), jnp.float32),
            pltpu.VMEM((CHUNK,), jnp.float32),
            pltpu.VMEM((CHUNK,), jnp.float32),
            pltpu.VMEM((CHUNK,), jnp.float32),
            pltpu.VMEM((CHUNK,), jnp.float32),
            pltpu.VMEM((16,), jnp.float32),
            pltpu.SemaphoreType.DMA,
            pltpu.SemaphoreType.DMA,
            pltpu.SemaphoreType.DMA,
            pltpu.SemaphoreType.DMA,
            pltpu.SemaphoreType.DMA,
        ],
    )(_sc_kernel)
    return kern(pred_flat, ytrue)


def kernel(y_pred, y_true):
    partial = _topk_hits(y_pred.reshape(-1), y_true.astype(jnp.int32))
    return partial[:, 0].sum() / jnp.float32(B)
